# baseline (speedup 1.0000x reference)
import jax
import jax.numpy as jnp
from jax import lax
from jax.experimental import pallas as pl
from jax.experimental.pallas import tpu as pltpu
from jax.experimental.pallas import tpu_sc as plsc

NC = 1
NS = 16
NW = NC * NS
LANES = 16


def _sc_body(f_in, f_out, din_hbm, w_hbm, b_hbm, out_hbm, x_v, o_v, w_s, b_s):
    n = din_hbm.shape[0]
    rows_per_w = n // f_in // NW
    words_per_w = rows_per_w * f_in
    wid = lax.axis_index("s") * NC + lax.axis_index("c")
    base = wid * words_per_w

    pltpu.sync_copy(din_hbm.at[pl.ds(base, words_per_w)], x_v)
    pltpu.sync_copy(w_hbm, w_s)
    pltpu.sync_copy(b_hbm, b_s)

    lanes32 = lax.iota(jnp.int32, LANES) * f_in
    bvecs = [b_s[pl.ds(h * LANES, LANES)] for h in range(f_in // LANES)]

    def group(g, carry):
        gbase = lanes32 + g * (LANES * f_in)
        acc = [jnp.zeros((LANES,), jnp.float32) for _ in range(f_out)]
        for j in range(f_in):
            xj = plsc.load_gather(x_v, [gbase + j])
            xb = xj + bvecs[j // LANES][j % LANES]
            wv = [w_s[pl.ds(j * f_out + h * LANES, LANES)]
                  for h in range(f_out // LANES)]
            for k in range(f_out):
                acc[k] = acc[k] + xb * wv[k // LANES][k % LANES]
        for k in range(f_out):
            plsc.store_scatter(o_v, [gbase + k], jnp.maximum(acc[k], 0.0))
        return carry

    lax.fori_loop(0, rows_per_w // LANES, group, 0)
    pltpu.sync_copy(o_v, out_hbm.at[pl.ds(base, words_per_w)])


@jax.jit
def kernel(din, weight, bias):
    b, f_in = din.shape
    f_out = weight.shape[1]
    words_per_w = b * f_in // NW
    mesh = plsc.VectorSubcoreMesh(
        core_axis_name="c", subcore_axis_name="s",
        num_cores=NC, num_subcores=NS)
    f = pl.kernel(
        lambda *refs: _sc_body(f_in, f_out, *refs),
        out_type=jax.ShapeDtypeStruct((b * f_out,), jnp.float32),
        mesh=mesh,
        scratch_types=[
            pltpu.VMEM((words_per_w,), jnp.float32),
            pltpu.VMEM((words_per_w,), jnp.float32),
            pltpu.VMEM((f_in * f_out,), jnp.float32),
            pltpu.VMEM((f_in,), jnp.float32),
        ],
        compiler_params=pltpu.CompilerParams(needs_layout_passes=False),
    )
    out = f(din.reshape(-1), weight.reshape(-1), bias)
    return out.reshape(b, f_out)


# speedup vs baseline: 1.4209x; 1.4209x over previous
import jax
import jax.numpy as jnp
from jax import lax
from jax.experimental import pallas as pl
from jax.experimental.pallas import tpu as pltpu
from jax.experimental.pallas import tpu_sc as plsc

NC = 2
NS = 16
NW = NC * NS
LANES = 16


def _sc_body(f_in, f_out, din_hbm, w_hbm, b_hbm, out_hbm, x_v, o_v, w_s, b_s):
    n = din_hbm.shape[0]
    rows_per_w = n // f_in // NW
    words_per_w = rows_per_w * f_in
    wid = lax.axis_index("s") * NC + lax.axis_index("c")
    base = wid * words_per_w

    pltpu.sync_copy(din_hbm.at[pl.ds(base, words_per_w)], x_v)
    pltpu.sync_copy(w_hbm, w_s)
    pltpu.sync_copy(b_hbm, b_s)

    lanes32 = lax.iota(jnp.int32, LANES) * f_in
    bvecs = [b_s[pl.ds(h * LANES, LANES)] for h in range(f_in // LANES)]

    def group(g, carry):
        gbase = lanes32 + g * (LANES * f_in)
        acc = [jnp.zeros((LANES,), jnp.float32) for _ in range(f_out)]
        for j in range(f_in):
            xj = plsc.load_gather(x_v, [gbase + j])
            xb = xj + bvecs[j // LANES][j % LANES]
            wv = [w_s[pl.ds(j * f_out + h * LANES, LANES)]
                  for h in range(f_out // LANES)]
            for k in range(f_out):
                acc[k] = acc[k] + xb * wv[k // LANES][k % LANES]
        for k in range(f_out):
            plsc.store_scatter(o_v, [gbase + k], jnp.maximum(acc[k], 0.0))
        return carry

    lax.fori_loop(0, rows_per_w // LANES, group, 0)
    pltpu.sync_copy(o_v, out_hbm.at[pl.ds(base, words_per_w)])


@jax.jit
def kernel(din, weight, bias):
    b, f_in = din.shape
    f_out = weight.shape[1]
    words_per_w = b * f_in // NW
    mesh = plsc.VectorSubcoreMesh(
        core_axis_name="c", subcore_axis_name="s",
        num_cores=NC, num_subcores=NS)
    f = pl.kernel(
        lambda *refs: _sc_body(f_in, f_out, *refs),
        out_type=jax.ShapeDtypeStruct((b * f_out,), jnp.float32),
        mesh=mesh,
        scratch_types=[
            pltpu.VMEM((words_per_w,), jnp.float32),
            pltpu.VMEM((words_per_w,), jnp.float32),
            pltpu.VMEM((f_in * f_out,), jnp.float32),
            pltpu.VMEM((f_in,), jnp.float32),
        ],
        compiler_params=pltpu.CompilerParams(needs_layout_passes=False, skip_device_barrier=True),
    )
    out = f(din.reshape(-1), weight.reshape(-1), bias)
    return out.reshape(b, f_out)


# speedup vs baseline: 1.6210x; 1.1408x over previous
import jax
import jax.numpy as jnp
from jax import lax
from jax.experimental import pallas as pl
from jax.experimental.pallas import tpu as pltpu
from jax.experimental.pallas import tpu_sc as plsc

NC = 2
NS = 16
NW = NC * NS
LANES = 16


def _sc_body(din_hbm, w_hbm, b_hbm, out_hbm, x_v, w_s, b_s):
    b = din_hbm.shape[0]
    f_in = din_hbm.shape[1]
    f_out = w_hbm.shape[1]
    rows_per_w = b // NW
    wid = lax.axis_index("s") * NC + lax.axis_index("c")
    base = wid * rows_per_w

    pltpu.sync_copy(din_hbm.at[pl.ds(base, rows_per_w)], x_v)
    pltpu.sync_copy(w_hbm, w_s)
    pltpu.sync_copy(b_hbm, b_s)

    lanes = lax.iota(jnp.int32, LANES)
    bvecs = [b_s[pl.ds(h * LANES, LANES)] for h in range(f_in // LANES)]

    def group(g, carry):
        rows = g * LANES + lanes
        acc = [jnp.zeros((LANES,), jnp.float32) for _ in range(f_out)]
        for j in range(f_in):
            xj = plsc.load_gather(
                x_v, [rows, jnp.full((LANES,), j, jnp.int32)])
            xb = xj + bvecs[j // LANES][j % LANES]
            wv = [w_s[j, pl.ds(h * LANES, LANES)]
                  for h in range(f_out // LANES)]
            for k in range(f_out):
                acc[k] = acc[k] + xb * wv[k // LANES][k % LANES]
        for k in range(f_out):
            plsc.store_scatter(
                x_v, [rows, jnp.full((LANES,), k, jnp.int32)],
                jnp.maximum(acc[k], 0.0))
        return carry

    lax.fori_loop(0, rows_per_w // LANES, group, 0)
    pltpu.sync_copy(x_v, out_hbm.at[pl.ds(base, rows_per_w)])


@jax.jit
def kernel(din, weight, bias):
    b, f_in = din.shape
    f_out = weight.shape[1]
    rows_per_w = b // NW
    mesh = plsc.VectorSubcoreMesh(
        core_axis_name="c", subcore_axis_name="s",
        num_cores=NC, num_subcores=NS)
    f = pl.kernel(
        _sc_body,
        out_type=jax.ShapeDtypeStruct((b, f_out), jnp.float32),
        mesh=mesh,
        scratch_types=[
            pltpu.VMEM((rows_per_w, f_in), jnp.float32),
            pltpu.VMEM((f_in, f_out), jnp.float32),
            pltpu.VMEM((f_in,), jnp.float32),
        ],
        compiler_params=pltpu.CompilerParams(needs_layout_passes=False),
    )
    return f(din, weight, bias)


# speedup vs baseline: 2.6309x; 1.6230x over previous
import jax
import jax.numpy as jnp
from jax import lax
from jax.experimental import pallas as pl
from jax.experimental.pallas import tpu as pltpu
from jax.experimental.pallas import tpu_sc as plsc

NC = 2
NS = 16
NW = NC * NS
LANES = 16


def _sc_body(dint_hbm, w_hbm, b_hbm, outt_hbm, x_v, w_s, b_s):
    f_in, b = dint_hbm.shape
    f_out = w_hbm.shape[1]
    cols_per_w = b // NW
    wid = lax.axis_index("s") * NC + lax.axis_index("c")
    base = wid * cols_per_w

    pltpu.sync_copy(dint_hbm.at[:, pl.ds(base, cols_per_w)], x_v)
    pltpu.sync_copy(w_hbm, w_s)
    pltpu.sync_copy(b_hbm, b_s)

    bvecs = [b_s[pl.ds(h * LANES, LANES)] for h in range(f_in // LANES)]

    def group(g, carry):
        col = g * LANES
        acc = [jnp.zeros((LANES,), jnp.float32) for _ in range(f_out)]
        for j in range(f_in):
            xj = x_v[j, pl.ds(col, LANES)]
            xb = xj + bvecs[j // LANES][j % LANES]
            wv = [w_s[j, pl.ds(h * LANES, LANES)]
                  for h in range(f_out // LANES)]
            for k in range(f_out):
                acc[k] = acc[k] + xb * wv[k // LANES][k % LANES]
        for k in range(f_out):
            x_v[k, pl.ds(col, LANES)] = jnp.maximum(acc[k], 0.0)
        return carry

    lax.fori_loop(0, cols_per_w // LANES, group, 0)
    pltpu.sync_copy(x_v, outt_hbm.at[:, pl.ds(base, cols_per_w)])


@jax.jit
def kernel(din, weight, bias):
    b, f_in = din.shape
    f_out = weight.shape[1]
    cols_per_w = b // NW
    mesh = plsc.VectorSubcoreMesh(
        core_axis_name="c", subcore_axis_name="s",
        num_cores=NC, num_subcores=NS)
    f = pl.kernel(
        _sc_body,
        out_type=jax.ShapeDtypeStruct((f_out, b), jnp.float32),
        mesh=mesh,
        scratch_types=[
            pltpu.VMEM((f_in, cols_per_w), jnp.float32),
            pltpu.VMEM((f_in, f_out), jnp.float32),
            pltpu.VMEM((f_in,), jnp.float32),
        ],
        compiler_params=pltpu.CompilerParams(needs_layout_passes=False),
    )
    return f(din.T, weight, bias).T


# speedup vs baseline: 2.7783x; 1.0560x over previous
import jax
import jax.numpy as jnp
from jax import lax
from jax.experimental import pallas as pl
from jax.experimental.pallas import tpu as pltpu
from jax.experimental.pallas import tpu_sc as plsc

NC = 2
NS = 16
NW = NC * NS
LANES = 16


def _sc_body(dint_hbm, w_hbm, b_hbm, outt_hbm, x_v, w_s, b_s, wb_v):
    f_in, b = dint_hbm.shape
    f_out = w_hbm.shape[1]
    cols_per_w = b // NW
    wid = lax.axis_index("s") * NC + lax.axis_index("c")
    base = wid * cols_per_w

    pltpu.sync_copy(dint_hbm.at[:, pl.ds(base, cols_per_w)], x_v)
    pltpu.sync_copy(w_hbm, w_s)
    pltpu.sync_copy(b_hbm, b_s)

    bvecs = [b_s[pl.ds(h * LANES, LANES)] for h in range(f_in // LANES)]
    nh = f_out // LANES

    def expand(j, carry):
        wv = [w_s[j, pl.ds(h * LANES, LANES)] for h in range(nh)]
        for k in range(f_out):
            wb_v[pl.ds(j * (f_out * LANES) + k * LANES, LANES)] = (
                jnp.full((LANES,), wv[k // LANES][k % LANES], jnp.float32))
        return carry

    lax.fori_loop(0, f_in, expand, 0)

    def group(g, carry):
        col = g * LANES
        acc = [jnp.zeros((LANES,), jnp.float32) for _ in range(f_out)]
        for j in range(f_in):
            xj = x_v[j, pl.ds(col, LANES)]
            xb = xj + bvecs[j // LANES][j % LANES]
            wv = [w_s[j, pl.ds(h * LANES, LANES)]
                  for h in range(nh)]
            for k in range(f_out):
                if k % 2 == 0:
                    wk = wb_v[pl.ds(j * (f_out * LANES) + k * LANES, LANES)]
                    acc[k] = acc[k] + xb * wk
                else:
                    acc[k] = acc[k] + xb * wv[k // LANES][k % LANES]
        for k in range(f_out):
            x_v[k, pl.ds(col, LANES)] = jnp.maximum(acc[k], 0.0)
        return carry

    lax.fori_loop(0, cols_per_w // LANES, group, 0)
    pltpu.sync_copy(x_v, outt_hbm.at[:, pl.ds(base, cols_per_w)])


@jax.jit
def kernel(din, weight, bias):
    b, f_in = din.shape
    f_out = weight.shape[1]
    cols_per_w = b // NW
    mesh = plsc.VectorSubcoreMesh(
        core_axis_name="c", subcore_axis_name="s",
        num_cores=NC, num_subcores=NS)
    f = pl.kernel(
        _sc_body,
        out_type=jax.ShapeDtypeStruct((f_out, b), jnp.float32),
        mesh=mesh,
        scratch_types=[
            pltpu.VMEM((f_in, cols_per_w), jnp.float32),
            pltpu.VMEM((f_in, f_out), jnp.float32),
            pltpu.VMEM((f_in,), jnp.float32),
            pltpu.VMEM((f_in * f_out * LANES,), jnp.float32),
        ],
        compiler_params=pltpu.CompilerParams(needs_layout_passes=False),
    )
    return f(din.T, weight, bias).T
